# baseline (speedup 1.0000x reference)
import functools

import numpy as np

import jax
import jax.numpy as jnp
from jax import lax
from jax.experimental import pallas as pl
from jax.experimental.pallas import tpu as pltpu
from jax.experimental.pallas import tpu_sc as plsc

N, E, D, DE = 10000, 320000, 128, 16
F = 2 * D
H = D // 2
EPS = 1e-5
NC, NS = 2, 16
NW = NC * NS
C = 80
L = 16
RA = 632
RB = N - (NS - 1) * RA
HMASK = np.int32(-65536)
HALF = np.int32(0x8000)
LMASK = np.int32(0xFFFF)

PARTS = (115200, 102400, 102400)
BE = 6400

_MESH = plsc.VectorSubcoreMesh(
    core_axis_name="c", subcore_axis_name="s", num_cores=NC, num_subcores=NS)


def _pack_bf16(ev, od):
    lbc = lax.bitcast_convert_type
    we = jnp.bitwise_and(
        jnp.right_shift(lbc(ev, jnp.int32) + HALF, 16), LMASK)
    wo = jnp.bitwise_and(lbc(od, jnp.int32) + HALF, HMASK)
    return jnp.bitwise_or(we, wo)


def _pq_body(a_ref, wse_ref, wso_ref, wde_ref, wdo_ref, p_ref, q_ref):
    a = a_ref[...]

    def mm(w_ref):
        return jnp.dot(a, w_ref[...], preferred_element_type=jnp.float32)

    p_ref[...] = _pack_bf16(mm(wse_ref), mm(wso_ref))
    q_ref[...] = _pack_bf16(mm(wde_ref), mm(wdo_ref))


def _pq(atom, wse, wso, wde, wdo):
    BN = 2000
    return pl.pallas_call(
        _pq_body,
        grid=(N // BN,),
        in_specs=[pl.BlockSpec((BN, D), lambda i: (i, 0))]
        + [pl.BlockSpec((D, D), lambda i: (0, 0))] * 4,
        out_specs=[pl.BlockSpec((BN, D), lambda i: (i, 0)),
                   pl.BlockSpec((BN, D), lambda i: (i, 0))],
        out_shape=[jax.ShapeDtypeStruct((N, D), jnp.int32),
                   jax.ShapeDtypeStruct((N, D), jnp.int32)],
    )(atom, wse, wso, wde, wdo)


def _make_edge_k(epw, nch):
    @functools.partial(
        pl.kernel,
        out_type=jax.ShapeDtypeStruct((epw * NW, D), jnp.int32),
        mesh=_MESH,
        scratch_types=[pltpu.VMEM((nch, C), jnp.int32),
                       pltpu.VMEM((nch, C), jnp.int32),
                       pltpu.VMEM((C, D), jnp.int32),
                       pltpu.VMEM((C, D), jnp.int32),
                       pltpu.VMEM((C, D), jnp.int32),
                       pltpu.VMEM((C, D), jnp.int32),
                       pltpu.VMEM((C, D), jnp.int32),
                       pltpu.VMEM((C, D), jnp.int32),
                       pltpu.SemaphoreType.DMA,
                       pltpu.SemaphoreType.DMA,
                       pltpu.SemaphoreType.DMA,
                       pltpu.SemaphoreType.DMA,
                       pltpu.SemaphoreType.DMA,
                       pltpu.SemaphoreType.DMA],
    )
    def _edge_k(p_hbm, q_hbm, src_hbm, dst_hbm, s_out,
                sidx, didx, pga, qga, sba, pgb, qgb, sbb,
                spa, sqa, ssa, spb, sqb, ssb):
        wid = lax.axis_index("s") * NC + lax.axis_index("c")
        base0 = wid * epw
        pltpu.sync_copy(src_hbm.at[wid], sidx)
        pltpu.sync_copy(dst_hbm.at[wid], didx)
        lbc = lax.bitcast_convert_type

        def wait_gather(pg, qg, sp, sq):
            pltpu.make_async_copy(p_hbm.at[pl.ds(0, C)], pg, sp).wait()
            pltpu.make_async_copy(q_hbm.at[pl.ds(0, C)], qg, sq).wait()

        def wait_store(sb, ss):
            pltpu.make_async_copy(sb, s_out.at[pl.ds(base0, C)], ss).wait()

        def compute(pg, qg, sbuf):
            def row_body(r, a2):
                for g in range(D // L):
                    sl = pl.ds(g * L, L)
                    aw = pg[r, sl]
                    bw = qg[r, sl]
                    se = (lbc(jnp.left_shift(aw, 16), jnp.float32)
                          + lbc(jnp.left_shift(bw, 16), jnp.float32))
                    so = (lbc(jnp.bitwise_and(aw, HMASK), jnp.float32)
                          + lbc(jnp.bitwise_and(bw, HMASK), jnp.float32))
                    we = jnp.bitwise_and(
                        jnp.right_shift(lbc(se, jnp.int32) + HALF, 16),
                        LMASK)
                    wo = jnp.bitwise_and(lbc(so, jnp.int32) + HALF, HMASK)
                    sbuf[r, sl] = jnp.bitwise_or(we, wo)
                return a2

            lax.fori_loop(0, C, row_body, 0)

        pltpu.async_copy(p_hbm.at[sidx.at[0]], pga, spa)
        pltpu.async_copy(q_hbm.at[didx.at[0]], qga, sqa)
        pltpu.async_copy(p_hbm.at[sidx.at[1]], pgb, spb)
        pltpu.async_copy(q_hbm.at[didx.at[1]], qgb, sqb)

        def pair_body(k, acc):
            i0 = 2 * k
            i1 = 2 * k + 1
            wait_gather(pga, qga, spa, sqa)

            @pl.when(k > 0)
            def _():
                wait_store(sba, ssa)

            compute(pga, qga, sba)
            pltpu.async_copy(sba, s_out.at[pl.ds(base0 + i0 * C, C)], ssa)

            @pl.when(i0 + 2 < nch)
            def _():
                pltpu.async_copy(p_hbm.at[sidx.at[i0 + 2]], pga, spa)
                pltpu.async_copy(q_hbm.at[didx.at[i0 + 2]], qga, sqa)

            wait_gather(pgb, qgb, spb, sqb)

            @pl.when(k > 0)
            def _():
                wait_store(sbb, ssb)

            compute(pgb, qgb, sbb)
            pltpu.async_copy(sbb, s_out.at[pl.ds(base0 + i1 * C, C)], ssb)

            @pl.when(i1 + 2 < nch)
            def _():
                pltpu.async_copy(p_hbm.at[sidx.at[i1 + 2]], pgb, spb)
                pltpu.async_copy(q_hbm.at[didx.at[i1 + 2]], qgb, sqb)

            return acc

        lax.fori_loop(0, nch // 2, pair_body, 0)
        if nch % 2 == 1:
            wait_gather(pga, qga, spa, sqa)
            wait_store(sba, ssa)
            compute(pga, qga, sba)
            pltpu.async_copy(sba, s_out.at[pl.ds(base0 + (nch - 1) * C, C)],
                             ssa)
        wait_store(sba, ssa)
        wait_store(sbb, ssb)

    return _edge_k


_edge_ka = _make_edge_k(PARTS[0] // NW, PARTS[0] // NW // C)
_edge_kb = _make_edge_k(PARTS[1] // NW, PARTS[1] // NW // C)


def _unpack_t(s_ref, nbr_ref, wne_ref, wno_ref, be_ref, bo_ref):
    sw = s_ref[...]
    lo = lax.bitcast_convert_type(jnp.left_shift(sw, 16), jnp.float32)
    hi = lax.bitcast_convert_type(jnp.bitwise_and(sw, HMASK), jnp.float32)
    nbr_t = nbr_ref[...]
    dn = (((0,), (0,)), ((), ()))
    te = lo + lax.dot_general(nbr_t, wne_ref[...], dn,
                              preferred_element_type=jnp.float32) + be_ref[...]
    to = hi + lax.dot_general(nbr_t, wno_ref[...], dn,
                              preferred_element_type=jnp.float32) + bo_ref[...]
    return te, to


def _stats_body(s_ref, nbr_ref, wne_ref, wno_ref, be_ref, bo_ref, o_ref):
    te, to = _unpack_t(s_ref, nbr_ref, wne_ref, wno_ref, be_ref, bo_ref)
    part = jnp.concatenate(
        [jnp.sum(te, axis=0), jnp.sum(to, axis=0),
         jnp.sum(te * te, axis=0), jnp.sum(to * to, axis=0)], axis=0)[None, :]

    @pl.when(pl.program_id(0) == 0)
    def _():
        o_ref[...] = part

    @pl.when(pl.program_id(0) != 0)
    def _():
        o_ref[...] += part


def _stats(s2d, nbr_t, wne, wno, be, bo, off):
    ex = s2d.shape[0]
    return pl.pallas_call(
        _stats_body,
        grid=(ex // BE,),
        in_specs=[pl.BlockSpec((BE, D), lambda i: (i, 0)),
                  pl.BlockSpec((DE, BE), lambda i: (0, i + off)),
                  pl.BlockSpec((DE, D), lambda i: (0, 0)),
                  pl.BlockSpec((DE, D), lambda i: (0, 0)),
                  pl.BlockSpec((1, D), lambda i: (0, 0)),
                  pl.BlockSpec((1, D), lambda i: (0, 0))],
        out_specs=pl.BlockSpec((1, 4 * D), lambda i: (0, 0)),
        out_shape=jax.ShapeDtypeStruct((1, 4 * D), jnp.float32),
    )(s2d, nbr_t, wne, wno, be, bo)


def _gate_body(s_ref, nbr_ref, wne_ref, wno_ref, be_ref, bo_ref,
               ae_ref, ao_ref, ce_ref, co_ref, o_ref):
    te, to = _unpack_t(s_ref, nbr_ref, wne_ref, wno_ref, be_ref, bo_ref)
    te = te * ae_ref[...] + ce_ref[...]
    to = to * ao_ref[...] + co_ref[...]

    def _sig(x):
        return 0.5 * jnp.tanh(0.5 * x) + 0.5

    def _sp(x):
        return x + jnp.log(1.0 + jnp.exp(-x))

    msg_e = _sig(te[:, :H]) * _sp(te[:, H:])
    msg_o = _sig(to[:, :H]) * _sp(to[:, H:])
    o_ref[...] = jnp.concatenate([msg_e, msg_o], axis=1)


def _gate(s2d, nbr_t, wne, wno, be, bo, ae, ao, ce, co, off):
    ex = s2d.shape[0]
    return pl.pallas_call(
        _gate_body,
        grid=(ex // BE,),
        in_specs=[pl.BlockSpec((BE, D), lambda i: (i, 0)),
                  pl.BlockSpec((DE, BE), lambda i: (0, i + off)),
                  pl.BlockSpec((DE, D), lambda i: (0, 0)),
                  pl.BlockSpec((DE, D), lambda i: (0, 0))]
        + [pl.BlockSpec((1, D), lambda i: (0, 0))] * 6,
        out_specs=pl.BlockSpec((BE, D), lambda i: (i, 0)),
        out_shape=jax.ShapeDtypeStruct((ex, D), jnp.float32),
    )(s2d, nbr_t, wne, wno, be, bo, ae, ao, ce, co)


def _make_scatter_k(epw, nch):
    @functools.partial(
        pl.kernel,
        out_type=[jax.ShapeDtypeStruct((N, D), jnp.float32),
                  jax.ShapeDtypeStruct((N, D), jnp.float32)],
        mesh=_MESH,
        scratch_types=[pltpu.VMEM((nch, C), jnp.int32),
                       pltpu.VMEM((C, D), jnp.float32),
                       pltpu.VMEM((C, D), jnp.float32),
                       pltpu.VMEM_SHARED((N, D), jnp.float32),
                       pltpu.SemaphoreType.DMA,
                       pltpu.SemaphoreType.DMA],
    )
    def _scatter_k(msg_hbm, dst_hbm, init0, init1, out0, out1,
                   didx, msga, msgb, shared, sla, slb):
        cid = lax.axis_index("c")
        sid = lax.axis_index("s")
        wid = sid * NC + cid
        rbase = sid * RA
        pltpu.sync_copy(dst_hbm.at[wid], didx)

        def init_from(src):
            @pl.when(sid < NS - 1)
            def _():
                pltpu.sync_copy(src.at[pl.ds(rbase, RA)],
                                shared.at[pl.ds(rbase, RA)])

            @pl.when(sid == NS - 1)
            def _():
                pltpu.sync_copy(src.at[pl.ds((NS - 1) * RA, RB)],
                                shared.at[pl.ds((NS - 1) * RA, RB)])

        @pl.when(cid == 0)
        def _():
            init_from(init0)

        @pl.when(cid == 1)
        def _():
            init_from(init1)

        plsc.subcore_barrier()
        base0 = wid * epw

        def wait_load(msgx, sl):
            pltpu.make_async_copy(msg_hbm.at[pl.ds(0, C)], msgx, sl).wait()

        pltpu.async_copy(msg_hbm.at[pl.ds(base0, C)], msga, sla)
        pltpu.async_copy(msg_hbm.at[pl.ds(base0 + C, C)], msgb, slb)

        def pair(k, acc):
            i0 = 2 * k
            i1 = 2 * k + 1
            wait_load(msga, sla)
            pltpu.sync_copy(msga, shared.at[didx.at[i0]], add=True)

            @pl.when(i0 + 2 < nch)
            def _():
                pltpu.async_copy(msg_hbm.at[pl.ds(base0 + (i0 + 2) * C, C)],
                                 msga, sla)

            wait_load(msgb, slb)
            pltpu.sync_copy(msgb, shared.at[didx.at[i1]], add=True)

            @pl.when(i1 + 2 < nch)
            def _():
                pltpu.async_copy(msg_hbm.at[pl.ds(base0 + (i1 + 2) * C, C)],
                                 msgb, slb)

            return acc

        lax.fori_loop(0, nch // 2, pair, 0)
        if nch % 2 == 1:
            wait_load(msga, sla)
            pltpu.sync_copy(msga, shared.at[didx.at[nch - 1]], add=True)
        plsc.subcore_barrier()

        @pl.when((cid == 0) & (sid < NS - 1))
        def _():
            pltpu.sync_copy(shared.at[pl.ds(rbase, RA)],
                            out0.at[pl.ds(rbase, RA)])

        @pl.when((cid == 0) & (sid == NS - 1))
        def _():
            pltpu.sync_copy(shared.at[pl.ds((NS - 1) * RA, RB)],
                            out0.at[pl.ds((NS - 1) * RA, RB)])

        @pl.when((cid == 1) & (sid < NS - 1))
        def _():
            pltpu.sync_copy(shared.at[pl.ds(rbase, RA)],
                            out1.at[pl.ds(rbase, RA)])

        @pl.when((cid == 1) & (sid == NS - 1))
        def _():
            pltpu.sync_copy(shared.at[pl.ds((NS - 1) * RA, RB)],
                            out1.at[pl.ds((NS - 1) * RA, RB)])

    return _scatter_k


_scatter_ka = _make_scatter_k(PARTS[0] // NW, PARTS[0] // NW // C)
_scatter_kb = _make_scatter_k(PARTS[1] // NW, PARTS[1] // NW // C)


def _final_body(u0_ref, u1_ref, atom_ref, g2_ref, b2_ref, o_ref):
    upd = u0_ref[...] + u1_ref[...]
    mean = jnp.mean(upd, axis=0, keepdims=True)
    var = jnp.mean((upd - mean) ** 2, axis=0, keepdims=True)
    nrm = (upd - mean) * lax.rsqrt(var + EPS) * g2_ref[...] + b2_ref[...]
    o_ref[...] = jax.nn.softplus(atom_ref[...] + nrm)


def _final(u0, u1, atom, g2, b2):
    return pl.pallas_call(
        _final_body,
        in_specs=[pl.BlockSpec((N, D), lambda: (0, 0))] * 3
        + [pl.BlockSpec((1, D), lambda: (0, 0))] * 2,
        out_specs=pl.BlockSpec((N, D), lambda: (0, 0)),
        out_shape=jax.ShapeDtypeStruct((N, D), jnp.float32),
    )(u0, u1, atom, g2, b2)


def kernel(atom_in_fea, nbr_fea, edge_src, edge_dst, W, b,
           g1, beta1, g2, beta2):
    ws = W[:, :D].T
    wd = W[:, D:2 * D].T
    wn = W[:, 2 * D:].T
    p_i, q_i = _pq(atom_in_fea, ws[:, 0::2], ws[:, 1::2],
                   wd[:, 0::2], wd[:, 1::2])

    edge_kernels = (_edge_ka, _edge_kb, _edge_kb)
    scatter_kernels = (_scatter_ka, _scatter_kb, _scatter_kb)
    bounds = [0]
    for pz in PARTS:
        bounds.append(bounds[-1] + pz)

    srcs, dsts, ss = [], [], []
    for i, pz in enumerate(PARTS):
        nchx = pz // NW // C
        srcs.append(edge_src[bounds[i]:bounds[i + 1]].reshape(NW, nchx, C))
        dsts.append(edge_dst[bounds[i]:bounds[i + 1]].reshape(NW, nchx, C))
        ss.append(edge_kernels[i](p_i, q_i, srcs[i], dsts[i]))

    nbr_t = nbr_fea.T
    wne = wn[:, 0::2]
    wno = wn[:, 1::2]
    be = b[0::2].reshape(1, D)
    bo = b[1::2].reshape(1, D)
    stats = sum(_stats(ss[i], nbr_t, wne, wno, be, bo, bounds[i] // BE)
                for i in range(len(PARTS)))

    rs = lax.rsqrt
    mean_e = stats[0, :D] / E
    mean_o = stats[0, D:2 * D] / E
    var_e = stats[0, 2 * D:3 * D] / E - mean_e * mean_e
    var_o = stats[0, 3 * D:] / E - mean_o * mean_o
    g1e, g1o = g1[0::2], g1[1::2]
    b1e, b1o = beta1[0::2], beta1[1::2]
    ae = (g1e * rs(var_e + EPS)).reshape(1, D)
    ao = (g1o * rs(var_o + EPS)).reshape(1, D)
    ce = (b1e - mean_e * g1e * rs(var_e + EPS)).reshape(1, D)
    co = (b1o - mean_o * g1o * rs(var_o + EPS)).reshape(1, D)

    msgs = [_gate(ss[i], nbr_t, wne, wno, be, bo, ae, ao, ce, co,
                  bounds[i] // BE) for i in range(len(PARTS))]

    u0 = jnp.zeros((N, D), jnp.float32)
    u1 = u0
    for i in range(len(PARTS)):
        u0, u1 = scatter_kernels[i](msgs[i], dsts[i], u0, u1)

    sigma = np.concatenate([np.arange(0, D, 2), np.arange(1, D, 2)])
    inv_sigma = np.argsort(sigma)
    atom_p = atom_in_fea[:, sigma]
    out_p = _final(u0, u1, atom_p,
                   g2[sigma].reshape(1, D), beta2[sigma].reshape(1, D))
    return out_p[:, inv_sigma]


# speedup vs baseline: 1.0011x; 1.0011x over previous
import functools

import numpy as np

import jax
import jax.numpy as jnp
from jax import lax
from jax.experimental import pallas as pl
from jax.experimental.pallas import tpu as pltpu
from jax.experimental.pallas import tpu_sc as plsc

N, E, D, DE = 10000, 320000, 128, 16
F = 2 * D
H = D // 2
EPS = 1e-5
NC, NS = 2, 16
NW = NC * NS
C = 80
L = 16
RA = 632
RB = N - (NS - 1) * RA
HMASK = np.int32(-65536)
HALF = np.int32(0x8000)
LMASK = np.int32(0xFFFF)

E1 = 192000
E2 = E - E1
BE = 6400

_MESH = plsc.VectorSubcoreMesh(
    core_axis_name="c", subcore_axis_name="s", num_cores=NC, num_subcores=NS)


def _pack_bf16(ev, od):
    lbc = lax.bitcast_convert_type
    we = jnp.bitwise_and(
        jnp.right_shift(lbc(ev, jnp.int32) + HALF, 16), LMASK)
    wo = jnp.bitwise_and(lbc(od, jnp.int32) + HALF, HMASK)
    return jnp.bitwise_or(we, wo)


def _pq_body(a_ref, wse_ref, wso_ref, wde_ref, wdo_ref, p_ref, q_ref):
    a = a_ref[...]

    def mm(w_ref):
        return jnp.dot(a, w_ref[...], preferred_element_type=jnp.float32)

    p_ref[...] = _pack_bf16(mm(wse_ref), mm(wso_ref))
    q_ref[...] = _pack_bf16(mm(wde_ref), mm(wdo_ref))


def _pq(atom, wse, wso, wde, wdo):
    BN = 2000
    return pl.pallas_call(
        _pq_body,
        grid=(N // BN,),
        in_specs=[pl.BlockSpec((BN, D), lambda i: (i, 0))]
        + [pl.BlockSpec((D, D), lambda i: (0, 0))] * 4,
        out_specs=[pl.BlockSpec((BN, D), lambda i: (i, 0)),
                   pl.BlockSpec((BN, D), lambda i: (i, 0))],
        out_shape=[jax.ShapeDtypeStruct((N, D), jnp.int32),
                   jax.ShapeDtypeStruct((N, D), jnp.int32)],
    )(atom, wse, wso, wde, wdo)


def _make_edge_k(epw, nch):
    @functools.partial(
        pl.kernel,
        out_type=jax.ShapeDtypeStruct((epw * NW, D), jnp.int32),
        mesh=_MESH,
        scratch_types=[pltpu.VMEM((nch, C), jnp.int32),
                       pltpu.VMEM((nch, C), jnp.int32),
                       pltpu.VMEM((C, D), jnp.int32),
                       pltpu.VMEM((C, D), jnp.int32),
                       pltpu.VMEM((C, D), jnp.int32),
                       pltpu.VMEM((C, D), jnp.int32),
                       pltpu.VMEM((C, D), jnp.int32),
                       pltpu.VMEM((C, D), jnp.int32),
                       pltpu.SemaphoreType.DMA,
                       pltpu.SemaphoreType.DMA,
                       pltpu.SemaphoreType.DMA,
                       pltpu.SemaphoreType.DMA,
                       pltpu.SemaphoreType.DMA,
                       pltpu.SemaphoreType.DMA],
    )
    def _edge_k(p_hbm, q_hbm, src_hbm, dst_hbm, s_out,
                sidx, didx, pga, qga, sba, pgb, qgb, sbb,
                spa, sqa, ssa, spb, sqb, ssb):
        wid = lax.axis_index("s") * NC + lax.axis_index("c")
        base0 = wid * epw
        pltpu.sync_copy(src_hbm.at[wid], sidx)
        pltpu.sync_copy(dst_hbm.at[wid], didx)
        lbc = lax.bitcast_convert_type

        def wait_gather(pg, qg, sp, sq):
            pltpu.make_async_copy(p_hbm.at[pl.ds(0, C)], pg, sp).wait()
            pltpu.make_async_copy(q_hbm.at[pl.ds(0, C)], qg, sq).wait()

        def wait_store(sb, ss):
            pltpu.make_async_copy(sb, s_out.at[pl.ds(base0, C)], ss).wait()

        def compute(pg, qg, sbuf):
            def row_body(r, a2):
                for g in range(D // L):
                    sl = pl.ds(g * L, L)
                    aw = pg[r, sl]
                    bw = qg[r, sl]
                    se = (lbc(jnp.left_shift(aw, 16), jnp.float32)
                          + lbc(jnp.left_shift(bw, 16), jnp.float32))
                    so = (lbc(jnp.bitwise_and(aw, HMASK), jnp.float32)
                          + lbc(jnp.bitwise_and(bw, HMASK), jnp.float32))
                    we = jnp.bitwise_and(
                        jnp.right_shift(lbc(se, jnp.int32) + HALF, 16),
                        LMASK)
                    wo = jnp.bitwise_and(lbc(so, jnp.int32) + HALF, HMASK)
                    sbuf[r, sl] = jnp.bitwise_or(we, wo)
                return a2

            lax.fori_loop(0, C, row_body, 0)

        pltpu.async_copy(p_hbm.at[sidx.at[0]], pga, spa)
        pltpu.async_copy(q_hbm.at[didx.at[0]], qga, sqa)
        pltpu.async_copy(p_hbm.at[sidx.at[1]], pgb, spb)
        pltpu.async_copy(q_hbm.at[didx.at[1]], qgb, sqb)

        def pair_body(k, acc):
            i0 = 2 * k
            i1 = 2 * k + 1
            wait_gather(pga, qga, spa, sqa)

            @pl.when(k > 0)
            def _():
                wait_store(sba, ssa)

            compute(pga, qga, sba)
            pltpu.async_copy(sba, s_out.at[pl.ds(base0 + i0 * C, C)], ssa)

            @pl.when(i0 + 2 < nch)
            def _():
                pltpu.async_copy(p_hbm.at[sidx.at[i0 + 2]], pga, spa)
                pltpu.async_copy(q_hbm.at[didx.at[i0 + 2]], qga, sqa)

            wait_gather(pgb, qgb, spb, sqb)

            @pl.when(k > 0)
            def _():
                wait_store(sbb, ssb)

            compute(pgb, qgb, sbb)
            pltpu.async_copy(sbb, s_out.at[pl.ds(base0 + i1 * C, C)], ssb)

            @pl.when(i1 + 2 < nch)
            def _():
                pltpu.async_copy(p_hbm.at[sidx.at[i1 + 2]], pgb, spb)
                pltpu.async_copy(q_hbm.at[didx.at[i1 + 2]], qgb, sqb)

            return acc

        lax.fori_loop(0, nch // 2, pair_body, 0)
        if nch % 2 == 1:
            wait_gather(pga, qga, spa, sqa)
            wait_store(sba, ssa)
            compute(pga, qga, sba)
            pltpu.async_copy(sba, s_out.at[pl.ds(base0 + (nch - 1) * C, C)],
                             ssa)
        wait_store(sba, ssa)
        wait_store(sbb, ssb)

    return _edge_k


_edge_k1 = _make_edge_k(E1 // NW, E1 // NW // C)
_edge_k2 = _make_edge_k(E2 // NW, E2 // NW // C)


def _unpack_t(s_ref, nbr_ref, wne_ref, wno_ref, be_ref, bo_ref):
    sw = s_ref[...]
    lo = lax.bitcast_convert_type(jnp.left_shift(sw, 16), jnp.float32)
    hi = lax.bitcast_convert_type(jnp.bitwise_and(sw, HMASK), jnp.float32)
    nbr_t = nbr_ref[...]
    dn = (((0,), (0,)), ((), ()))
    te = lo + lax.dot_general(nbr_t, wne_ref[...], dn,
                              preferred_element_type=jnp.float32) + be_ref[...]
    to = hi + lax.dot_general(nbr_t, wno_ref[...], dn,
                              preferred_element_type=jnp.float32) + bo_ref[...]
    return te, to


def _stats_body(s_ref, nbr_ref, wne_ref, wno_ref, be_ref, bo_ref, o_ref):
    te, to = _unpack_t(s_ref, nbr_ref, wne_ref, wno_ref, be_ref, bo_ref)
    part = jnp.concatenate(
        [jnp.sum(te, axis=0), jnp.sum(to, axis=0),
         jnp.sum(te * te, axis=0), jnp.sum(to * to, axis=0)], axis=0)[None, :]

    @pl.when(pl.program_id(0) == 0)
    def _():
        o_ref[...] = part

    @pl.when(pl.program_id(0) != 0)
    def _():
        o_ref[...] += part


def _stats(s2d, nbr_t, wne, wno, be, bo, off):
    ex = s2d.shape[0]
    return pl.pallas_call(
        _stats_body,
        grid=(ex // BE,),
        in_specs=[pl.BlockSpec((BE, D), lambda i: (i, 0)),
                  pl.BlockSpec((DE, BE), lambda i: (0, i + off)),
                  pl.BlockSpec((DE, D), lambda i: (0, 0)),
                  pl.BlockSpec((DE, D), lambda i: (0, 0)),
                  pl.BlockSpec((1, D), lambda i: (0, 0)),
                  pl.BlockSpec((1, D), lambda i: (0, 0))],
        out_specs=pl.BlockSpec((1, 4 * D), lambda i: (0, 0)),
        out_shape=jax.ShapeDtypeStruct((1, 4 * D), jnp.float32),
    )(s2d, nbr_t, wne, wno, be, bo)


def _gate_body(s_ref, nbr_ref, wne_ref, wno_ref, be_ref, bo_ref,
               ae_ref, ao_ref, ce_ref, co_ref, o_ref):
    te, to = _unpack_t(s_ref, nbr_ref, wne_ref, wno_ref, be_ref, bo_ref)
    te = te * ae_ref[...] + ce_ref[...]
    to = to * ao_ref[...] + co_ref[...]

    def _sig(x):
        return 0.5 * jnp.tanh(0.5 * x) + 0.5

    def _sp(x):
        return x + jnp.log(1.0 + jnp.exp(-x))

    msg_e = _sig(te[:, :H]) * _sp(te[:, H:])
    msg_o = _sig(to[:, :H]) * _sp(to[:, H:])
    o_ref[...] = jnp.concatenate([msg_e, msg_o], axis=1)


def _gate(s2d, nbr_t, wne, wno, be, bo, ae, ao, ce, co, off):
    ex = s2d.shape[0]
    return pl.pallas_call(
        _gate_body,
        grid=(ex // BE,),
        in_specs=[pl.BlockSpec((BE, D), lambda i: (i, 0)),
                  pl.BlockSpec((DE, BE), lambda i: (0, i + off)),
                  pl.BlockSpec((DE, D), lambda i: (0, 0)),
                  pl.BlockSpec((DE, D), lambda i: (0, 0))]
        + [pl.BlockSpec((1, D), lambda i: (0, 0))] * 6,
        out_specs=pl.BlockSpec((BE, D), lambda i: (i, 0)),
        out_shape=jax.ShapeDtypeStruct((ex, D), jnp.float32),
    )(s2d, nbr_t, wne, wno, be, bo, ae, ao, ce, co)


def _make_scatter_k(epw, nch):
    @functools.partial(
        pl.kernel,
        out_type=[jax.ShapeDtypeStruct((N, D), jnp.float32),
                  jax.ShapeDtypeStruct((N, D), jnp.float32)],
        mesh=_MESH,
        scratch_types=[pltpu.VMEM((nch, C), jnp.int32),
                       pltpu.VMEM((C, D), jnp.float32),
                       pltpu.VMEM((C, D), jnp.float32),
                       pltpu.VMEM_SHARED((N, D), jnp.float32),
                       pltpu.SemaphoreType.DMA,
                       pltpu.SemaphoreType.DMA],
    )
    def _scatter_k(msg_hbm, dst_hbm, init0, init1, out0, out1,
                   didx, msga, msgb, shared, sla, slb):
        cid = lax.axis_index("c")
        sid = lax.axis_index("s")
        wid = sid * NC + cid
        rbase = sid * RA
        pltpu.sync_copy(dst_hbm.at[wid], didx)

        def init_from(src):
            @pl.when(sid < NS - 1)
            def _():
                pltpu.sync_copy(src.at[pl.ds(rbase, RA)],
                                shared.at[pl.ds(rbase, RA)])

            @pl.when(sid == NS - 1)
            def _():
                pltpu.sync_copy(src.at[pl.ds((NS - 1) * RA, RB)],
                                shared.at[pl.ds((NS - 1) * RA, RB)])

        @pl.when(cid == 0)
        def _():
            init_from(init0)

        @pl.when(cid == 1)
        def _():
            init_from(init1)

        plsc.subcore_barrier()
        base0 = wid * epw

        def wait_load(msgx, sl):
            pltpu.make_async_copy(msg_hbm.at[pl.ds(0, C)], msgx, sl).wait()

        pltpu.async_copy(msg_hbm.at[pl.ds(base0, C)], msga, sla)
        pltpu.async_copy(msg_hbm.at[pl.ds(base0 + C, C)], msgb, slb)

        def pair(k, acc):
            i0 = 2 * k
            i1 = 2 * k + 1
            wait_load(msga, sla)
            pltpu.sync_copy(msga, shared.at[didx.at[i0]], add=True)

            @pl.when(i0 + 2 < nch)
            def _():
                pltpu.async_copy(msg_hbm.at[pl.ds(base0 + (i0 + 2) * C, C)],
                                 msga, sla)

            wait_load(msgb, slb)
            pltpu.sync_copy(msgb, shared.at[didx.at[i1]], add=True)

            @pl.when(i1 + 2 < nch)
            def _():
                pltpu.async_copy(msg_hbm.at[pl.ds(base0 + (i1 + 2) * C, C)],
                                 msgb, slb)

            return acc

        lax.fori_loop(0, nch // 2, pair, 0)
        if nch % 2 == 1:
            wait_load(msga, sla)
            pltpu.sync_copy(msga, shared.at[didx.at[nch - 1]], add=True)
        plsc.subcore_barrier()

        @pl.when((cid == 0) & (sid < NS - 1))
        def _():
            pltpu.sync_copy(shared.at[pl.ds(rbase, RA)],
                            out0.at[pl.ds(rbase, RA)])

        @pl.when((cid == 0) & (sid == NS - 1))
        def _():
            pltpu.sync_copy(shared.at[pl.ds((NS - 1) * RA, RB)],
                            out0.at[pl.ds((NS - 1) * RA, RB)])

        @pl.when((cid == 1) & (sid < NS - 1))
        def _():
            pltpu.sync_copy(shared.at[pl.ds(rbase, RA)],
                            out1.at[pl.ds(rbase, RA)])

        @pl.when((cid == 1) & (sid == NS - 1))
        def _():
            pltpu.sync_copy(shared.at[pl.ds((NS - 1) * RA, RB)],
                            out1.at[pl.ds((NS - 1) * RA, RB)])

    return _scatter_k


_scatter_k1 = _make_scatter_k(E1 // NW, E1 // NW // C)
_scatter_k2 = _make_scatter_k(E2 // NW, E2 // NW // C)


def _final_body(u0_ref, u1_ref, atom_ref, g2_ref, b2_ref, o_ref):
    upd = u0_ref[...] + u1_ref[...]
    mean = jnp.mean(upd, axis=0, keepdims=True)
    var = jnp.mean((upd - mean) ** 2, axis=0, keepdims=True)
    nrm = (upd - mean) * lax.rsqrt(var + EPS) * g2_ref[...] + b2_ref[...]
    o_ref[...] = jax.nn.softplus(atom_ref[...] + nrm)


def _final(u0, u1, atom, g2, b2):
    return pl.pallas_call(
        _final_body,
        in_specs=[pl.BlockSpec((N, D), lambda: (0, 0))] * 3
        + [pl.BlockSpec((1, D), lambda: (0, 0))] * 2,
        out_specs=pl.BlockSpec((N, D), lambda: (0, 0)),
        out_shape=jax.ShapeDtypeStruct((N, D), jnp.float32),
    )(u0, u1, atom, g2, b2)


def kernel(atom_in_fea, nbr_fea, edge_src, edge_dst, W, b,
           g1, beta1, g2, beta2):
    ws = W[:, :D].T
    wd = W[:, D:2 * D].T
    wn = W[:, 2 * D:].T
    p_i, q_i = _pq(atom_in_fea, ws[:, 0::2], ws[:, 1::2],
                   wd[:, 0::2], wd[:, 1::2])

    nch1 = E1 // NW // C
    nch2 = E2 // NW // C
    src1 = edge_src[:E1].reshape(NW, nch1, C)
    dst1 = edge_dst[:E1].reshape(NW, nch1, C)
    src2 = edge_src[E1:].reshape(NW, nch2, C)
    dst2 = edge_dst[E1:].reshape(NW, nch2, C)

    s1 = _edge_k1(p_i, q_i, src1, dst1)
    s2 = _edge_k2(p_i, q_i, src2, dst2)

    nbr_t = nbr_fea.T
    wne = wn[:, 0::2]
    wno = wn[:, 1::2]
    be = b[0::2].reshape(1, D)
    bo = b[1::2].reshape(1, D)
    st1 = _stats(s1, nbr_t, wne, wno, be, bo, 0)
    st2 = _stats(s2, nbr_t, wne, wno, be, bo, E1 // BE)
    stats = st1 + st2

    rs = lax.rsqrt
    mean_e = stats[0, :D] / E
    mean_o = stats[0, D:2 * D] / E
    var_e = stats[0, 2 * D:3 * D] / E - mean_e * mean_e
    var_o = stats[0, 3 * D:] / E - mean_o * mean_o
    g1e, g1o = g1[0::2], g1[1::2]
    b1e, b1o = beta1[0::2], beta1[1::2]
    ae = (g1e * rs(var_e + EPS)).reshape(1, D)
    ao = (g1o * rs(var_o + EPS)).reshape(1, D)
    ce = (b1e - mean_e * g1e * rs(var_e + EPS)).reshape(1, D)
    co = (b1o - mean_o * g1o * rs(var_o + EPS)).reshape(1, D)

    msg1 = _gate(s1, nbr_t, wne, wno, be, bo, ae, ao, ce, co, 0)
    msg2 = _gate(s2, nbr_t, wne, wno, be, bo, ae, ao, ce, co, E1 // BE)

    zeros = jnp.zeros((N, D), jnp.float32)
    v0, v1 = _scatter_k1(msg1, dst1, zeros, zeros)
    u0, u1 = _scatter_k2(msg2, dst2, v0, v1)

    sigma = np.concatenate([np.arange(0, D, 2), np.arange(1, D, 2)])
    inv_sigma = np.argsort(sigma)
    atom_p = atom_in_fea[:, sigma]
    out_p = _final(u0, u1, atom_p,
                   g2[sigma].reshape(1, D), beta2[sigma].reshape(1, D))
    return out_p[:, inv_sigma]
